# baseline (speedup 1.0000x reference)
import functools

import jax
import jax.numpy as jnp
from jax import lax
from jax.experimental import pallas as pl
from jax.experimental.pallas import tpu as pltpu
from jax.experimental.pallas import tpu_sc as plsc

EMBD = 64
NUM_CORES = 2
NUM_SUBCORES = 16
NW = NUM_CORES * NUM_SUBCORES
BLK = 128
L16 = 16
NBUF = 4


@jax.jit
def _sc_gather(idx, t2):
    n_items = idx.shape[0]
    per_w = n_items // NW
    blocks_per_w = per_w // BLK
    n_l_blocks = 32
    mesh = plsc.VectorSubcoreMesh(core_axis_name="c", subcore_axis_name="s")

    @functools.partial(
        pl.kernel,
        mesh=mesh,
        compiler_params=pltpu.CompilerParams(
            use_tc_tiling_on_sc=False, needs_layout_passes=False
        ),
        out_type=jax.ShapeDtypeStruct((n_items // L16, 8, 128), jnp.float32),
        scratch_types=[
            pltpu.VMEM((per_w,), jnp.int32),
            pltpu.VMEM((NBUF, BLK), jnp.int32),
            pltpu.VMEM((BLK, 128), jnp.float32),
            pltpu.VMEM((BLK, 128), jnp.float32),
            pltpu.VMEM((BLK, 128), jnp.float32),
            pltpu.VMEM((BLK, 128), jnp.float32),
            pltpu.VMEM((EMBD, 128), jnp.float32),
            pltpu.VMEM((EMBD, 128), jnp.float32),
            pltpu.SemaphoreType.DMA((NBUF,)),
            pltpu.SemaphoreType.DMA((2,)),
        ],
    )
    def body(
        idx_hbm, t2_hbm, out_hbm, xv, ring, st0, st1, st2, st3, asm0, asm1,
        gsem, ssem
    ):
        wid = lax.axis_index("s") * NUM_CORES + lax.axis_index("c")
        base = wid * per_w
        pltpu.sync_copy(idx_hbm.at[pl.ds(base, per_w)], xv)

        iota = lax.iota(jnp.int32, L16)
        staged = (st0, st1, st2, st3)
        asm = (asm0, asm1)
        first_block = wid * blocks_per_w

        def make_ids(n, p):
            for g in range(8):
                v = xv[pl.ds(n * BLK + g * L16, L16)]
                ring[p, pl.ds(g * L16, L16)] = lax.shift_right_logical(v, 1)

        def gather_desc(n, p):
            return pltpu.make_async_copy(
                t2_hbm.at[ring.at[p]], staged[p], gsem.at[p]
            )

        def store_desc(n, a, te):
            blk = first_block + n
            l = blk // n_l_blocks
            tb = blk % n_l_blocks
            m = (l * 8 + te) * n_l_blocks + tb
            return pltpu.make_async_copy(
                asm[a].at[pl.ds(te * 8, 8)],
                out_hbm.at[m],
                ssem.at[a],
            )

        def transpose_block(n, p, a):
            colb = [
                lax.shift_left(
                    lax.bitwise_and(xv[pl.ds(n * BLK + g * L16, L16)], 1), 6
                )
                for g in range(8)
            ]
            rows = [iota + g * L16 for g in range(8)]

            @plsc.parallel_loop(0, EMBD, unroll=8)
            def erow(e):
                for g in range(8):
                    val = plsc.load_gather(staged[p], [rows[g], colb[g] + e])
                    asm[a][e, pl.ds(g * L16, L16)] = val

        make_ids(0, 0)
        gather_desc(0, 0).start()
        make_ids(1, 1)
        gather_desc(1, 1).start()

        def step(n, p, a):
            @pl.when(n + 2 < blocks_per_w)
            def _():
                make_ids(n + 2, (p + 2) % NBUF)

            gather_desc(n, p).wait()

            @pl.when(n + 2 < blocks_per_w)
            def _():
                gather_desc(n + 2, (p + 2) % NBUF).start()

            @pl.when(n >= 2)
            def _():
                for te in range(8):
                    store_desc(n - 2, a, te).wait()

            transpose_block(n, p, a)
            for te in range(8):
                store_desc(n, a, te).start()

        def quad(k, carry):
            for j in range(NBUF):
                step(NBUF * k + j, j, j % 2)
            return carry

        lax.fori_loop(0, blocks_per_w // NBUF, quad, 0)

        for te in range(8):
            store_desc(blocks_per_w - 2, 0, te).wait()
            store_desc(blocks_per_w - 1, 1, te).wait()

    return body(idx, t2)


def kernel(x, table):
    b, l = x.shape
    idx_t = x.T.reshape(-1).astype(jnp.int32)
    t2 = table.reshape(table.shape[0] // 2, 2 * EMBD)
    out3 = _sc_gather(idx_t, t2)
    out5 = out3.reshape(l, 8, b // 128, 8, 128)
    return out5.transpose(2, 4, 0, 1, 3).reshape(b, l, EMBD)


# speedup vs baseline: 1.0082x; 1.0082x over previous
import functools

import jax
import jax.numpy as jnp
from jax import lax
from jax.experimental import pallas as pl
from jax.experimental.pallas import tpu as pltpu
from jax.experimental.pallas import tpu_sc as plsc

EMBD = 64
NUM_CORES = 2
NUM_SUBCORES = 16
NW = NUM_CORES * NUM_SUBCORES
BLK = 128
L16 = 16


@jax.jit
def _sc_gather(idx, t1):
    n_items = idx.shape[0]
    per_w = n_items // NW
    blocks_per_w = per_w // BLK
    n_l_blocks = 32
    mesh = plsc.VectorSubcoreMesh(core_axis_name="c", subcore_axis_name="s")

    @functools.partial(
        pl.kernel,
        mesh=mesh,
        compiler_params=pltpu.CompilerParams(
            use_tc_tiling_on_sc=False, needs_layout_passes=False
        ),
        out_type=jax.ShapeDtypeStruct((n_items // L16, 8, 128), jnp.float32),
        scratch_types=[
            pltpu.VMEM((per_w,), jnp.int32),
            pltpu.VMEM((BLK, EMBD), jnp.float32),
            pltpu.VMEM((BLK, EMBD), jnp.float32),
            pltpu.VMEM((BLK, EMBD), jnp.float32),
            pltpu.VMEM((BLK, EMBD), jnp.float32),
            pltpu.VMEM((EMBD, 128), jnp.float32),
            pltpu.VMEM((EMBD, 128), jnp.float32),
            pltpu.SemaphoreType.DMA((4,)),
            pltpu.SemaphoreType.DMA((2,)),
        ],
    )
    def body(
        idx_hbm, t1_hbm, out_hbm, xv, st0, st1, st2, st3, asm0, asm1,
        gsem, ssem
    ):
        wid = lax.axis_index("s") * NUM_CORES + lax.axis_index("c")
        base = wid * per_w
        pltpu.sync_copy(idx_hbm.at[pl.ds(base, per_w)], xv)

        iota = lax.iota(jnp.int32, L16)
        staged = (st0, st1, st2, st3)
        asm = (asm0, asm1)
        first_block = wid * blocks_per_w

        def gather_desc(n, p):
            return pltpu.make_async_copy(
                t1_hbm.at[xv.at[pl.ds(n * BLK, BLK)]],
                staged[p],
                gsem.at[p],
            )

        def store_desc(n, a, te):
            blk = first_block + n
            l = blk // n_l_blocks
            tb = blk % n_l_blocks
            m = (l * 8 + te) * n_l_blocks + tb
            return pltpu.make_async_copy(
                asm[a].at[pl.ds(te * 8, 8)],
                out_hbm.at[m],
                ssem.at[a],
            )

        def transpose_block(p, a):
            rows = [iota + g * L16 for g in range(8)]

            @plsc.parallel_loop(0, EMBD, unroll=8)
            def erow(e):
                col = jnp.broadcast_to(e, (L16,))
                for g in range(8):
                    val = plsc.load_gather(staged[p], [rows[g], col])
                    asm[a][e, pl.ds(g * L16, L16)] = val

        gather_desc(0, 0).start()
        gather_desc(1, 1).start()

        def step(n, p, a):
            gather_desc(n, p).wait()

            @pl.when(n + 2 < blocks_per_w)
            def _():
                gather_desc(n + 2, (p + 2) % 4).start()

            @pl.when(n >= 2)
            def _():
                for te in range(8):
                    store_desc(n - 2, a, te).wait()

            transpose_block(p, a)
            for te in range(8):
                store_desc(n, a, te).start()

        def quad(k, carry):
            for j in range(4):
                step(4 * k + j, j, j % 2)
            return carry

        lax.fori_loop(0, blocks_per_w // 4, quad, 0)

        for te in range(8):
            store_desc(blocks_per_w - 2, 0, te).wait()
            store_desc(blocks_per_w - 1, 1, te).wait()

    return body(idx, t1)


def kernel(x, table):
    b, l = x.shape
    idx_t = x.T.reshape(-1).astype(jnp.int32)
    out3 = _sc_gather(idx_t, table)
    out5 = out3.reshape(l, 8, b // 128, 8, 128)
    return out5.transpose(2, 4, 0, 1, 3).reshape(b, l, EMBD)


# speedup vs baseline: 1.7111x; 1.6971x over previous
import functools

import jax
import jax.numpy as jnp
from jax import lax
from jax.experimental import pallas as pl
from jax.experimental.pallas import tpu as pltpu
from jax.experimental.pallas import tpu_sc as plsc

EMBD = 64
NUM_CORES = 2
NUM_SUBCORES = 16
NW = NUM_CORES * NUM_SUBCORES
BLK = 128
L16 = 16


@jax.jit
def _sc_gather(idx, t1):
    n_items = idx.shape[0]
    per_w = n_items // NW
    blocks_per_w = per_w // BLK
    n_l_blocks = 32
    mesh = plsc.VectorSubcoreMesh(core_axis_name="c", subcore_axis_name="s")

    @functools.partial(
        pl.kernel,
        mesh=mesh,
        compiler_params=pltpu.CompilerParams(
            use_tc_tiling_on_sc=False, needs_layout_passes=False
        ),
        out_type=jax.ShapeDtypeStruct((n_items // L16, 8, 128), jnp.float32),
        scratch_types=[
            pltpu.VMEM((per_w,), jnp.int32),
            pltpu.VMEM((BLK, EMBD), jnp.float32),
            pltpu.VMEM((BLK, EMBD), jnp.float32),
            pltpu.VMEM((BLK, EMBD), jnp.float32),
            pltpu.VMEM((BLK, EMBD), jnp.float32),
            pltpu.VMEM((EMBD, 129), jnp.float32),
            pltpu.VMEM((EMBD, 129), jnp.float32),
            pltpu.SemaphoreType.DMA((4,)),
            pltpu.SemaphoreType.DMA((2,)),
        ],
    )
    def body(
        idx_hbm, t1_hbm, out_hbm, xv, st0, st1, st2, st3, asm0, asm1,
        gsem, ssem
    ):
        wid = lax.axis_index("s") * NUM_CORES + lax.axis_index("c")
        base = wid * per_w
        pltpu.sync_copy(idx_hbm.at[pl.ds(base, per_w)], xv)

        iota = lax.iota(jnp.int32, L16)
        staged = (st0, st1, st2, st3)
        asm = (asm0, asm1)
        first_block = wid * blocks_per_w

        def gather_desc(n, p):
            return pltpu.make_async_copy(
                t1_hbm.at[xv.at[pl.ds(n * BLK, BLK)]],
                staged[p],
                gsem.at[p],
            )

        def store_desc(n, a, te):
            blk = first_block + n
            l = blk // n_l_blocks
            tb = blk % n_l_blocks
            m = (l * 8 + te) * n_l_blocks + tb
            return pltpu.make_async_copy(
                asm[a].at[pl.ds(te * 8, 8), pl.ds(0, 128)],
                out_hbm.at[m],
                ssem.at[a],
            )

        def transpose_block(p, a):
            rowv = [iota + k * L16 for k in range(EMBD // L16)]

            @plsc.parallel_loop(0, BLK, unroll=4)
            def item(j):
                colv = jnp.broadcast_to(j, (L16,))
                for k in range(EMBD // L16):
                    val = staged[p][j, pl.ds(k * L16, L16)]
                    plsc.store_scatter(asm[a], [rowv[k], colv], val)

        gather_desc(0, 0).start()
        gather_desc(1, 1).start()

        def step(n, p, a):
            gather_desc(n, p).wait()

            @pl.when(n + 2 < blocks_per_w)
            def _():
                gather_desc(n + 2, (p + 2) % 4).start()

            @pl.when(n >= 2)
            def _():
                for te in range(8):
                    store_desc(n - 2, a, te).wait()

            transpose_block(p, a)
            for te in range(8):
                store_desc(n, a, te).start()

        def quad(k, carry):
            for j in range(4):
                step(4 * k + j, j, j % 2)
            return carry

        lax.fori_loop(0, blocks_per_w // 4, quad, 0)

        for te in range(8):
            store_desc(blocks_per_w - 2, 0, te).wait()
            store_desc(blocks_per_w - 1, 1, te).wait()

    return body(idx, t1)


def kernel(x, table):
    b, l = x.shape
    idx_t = x.T.reshape(-1).astype(jnp.int32)
    out3 = _sc_gather(idx_t, table)
    out5 = out3.reshape(l, 8, b // 128, 8, 128)
    return out5.transpose(2, 4, 0, 1, 3).reshape(b, l, EMBD)
